# baseline (speedup 1.0000x reference)
import functools

import jax
import jax.numpy as jnp
from jax import lax
from jax.experimental import pallas as pl
from jax.experimental.pallas import tpu as pltpu
from jax.experimental.pallas import tpu_sc as plsc

_NE = 8192
_D = 32
_H = 64
_A = 16
_BT = 256
_NCH = 64
_CW = 128
_NG = 8
_NC, _NS, _L = 2, 16, 16
_NW = _NC * _NS
_TPW = 1024 // _NW
_CAP = 2176


def _dist_chunks_body(r_ref, cb_ref, d3_ref, ids_ref, thr_ref, cm_ref):
    g = pl.program_id(0)
    r = r_ref[...]
    cb = cb_ref[...]
    rr = jnp.sum(r * r, axis=-1, keepdims=True)
    rc = lax.dot_general(r, cb, (((1,), (1,)), ((), ())),
                         preferred_element_type=jnp.float32)
    cc = jnp.sum(cb * cb, axis=-1)[None, :]
    d = (rr - 2.0 * rc) + cc
    mins = []
    for jj in range(_NG):
        slab = d[:, _CW * jj: _CW * (jj + 1)]
        d3_ref[jj] = slab
        mins.append(jnp.min(slab, axis=1, keepdims=True))
    cm_ref[g] = jnp.concatenate(mins, axis=1)

    @pl.when(g == _NG - 1)
    def _fin():
        cm = jnp.concatenate([cm_ref[i] for i in range(_NG)], axis=1)
        iota = lax.broadcasted_iota(jnp.int32, cm.shape, 1)
        inf = jnp.float32(float("inf"))
        cols = []
        m = None
        for _ in range(_A):
            m = jnp.min(cm, axis=1, keepdims=True)
            sel = jnp.min(jnp.where(cm == m, iota, _NCH), axis=1)
            cols.append(sel[:, None])
            cm = jnp.where(iota == sel[:, None], inf, cm)
        ids_ref[...] = jnp.concatenate(cols, axis=1)
        thr_ref[...] = jnp.broadcast_to(m, thr_ref.shape)


def _dist_chunks(r, cb):
    B = r.shape[0]
    return pl.pallas_call(
        _dist_chunks_body,
        grid=(_NG,),
        in_specs=[
            pl.BlockSpec((B, _D), lambda g: (0, 0)),
            pl.BlockSpec((_NE // _NG, _D), lambda g: (g, 0)),
        ],
        out_specs=[
            pl.BlockSpec((_NG, B, _CW), lambda g: (g, 0, 0)),
            pl.BlockSpec((B, _A), lambda g: (0, 0)),
            pl.BlockSpec((B, _A), lambda g: (0, 0)),
        ],
        out_shape=[
            jax.ShapeDtypeStruct((_NCH, B, _CW), jnp.float32),
            jax.ShapeDtypeStruct((B, _A), jnp.int32),
            jax.ShapeDtypeStruct((B, _A), jnp.float32),
        ],
        scratch_shapes=[pltpu.VMEM((_NG, B, _NG), jnp.float32)],
    )(r, cb)


def _splat_lane(vec, i):
    idx = jnp.full((_L,), i, jnp.int32)
    return lax.gather(
        vec, idx[:, None],
        lax.GatherDimensionNumbers(offset_dims=(), collapsed_slice_dims=(0,),
                                   start_index_map=(0,)),
        (1,), mode=lax.GatherScatterMode.PROMISE_IN_BOUNDS)


def _sc_topk_gather(dist_rows, ids_flat, thr_flat, cb_pad):
    mesh = plsc.VectorSubcoreMesh(core_axis_name="c", subcore_axis_name="s",
                                  num_cores=_NC, num_subcores=_NS)
    npt = _TPW * _A
    nbat = _TPW // 8

    @functools.partial(
        pl.kernel, mesh=mesh,
        compiler_params=pltpu.CompilerParams(needs_layout_passes=False),
        out_type=[
            jax.ShapeDtypeStruct((1024 * _A,), jnp.int32),
            jax.ShapeDtypeStruct((1024 * _A, _CW), jnp.float32),
        ],
        scratch_types=[
            pltpu.VMEM((npt,), jnp.int32),
            pltpu.VMEM((npt,), jnp.float32),
            pltpu.VMEM((nbat, 8 * _A), jnp.int32),
            pltpu.VMEM((8 * _A, _CW), jnp.float32),
            pltpu.VMEM((_CAP,), jnp.float32),
            pltpu.VMEM((_CAP,), jnp.int32),
            pltpu.VMEM((npt,), jnp.int32),
            pltpu.VMEM((npt, _CW), jnp.float32),
            pltpu.SemaphoreType.DMA,
        ],
    )
    def k(dist_hbm, ids_hbm, thr_hbm, cb_hbm, topk_hbm, codes_hbm,
          ids_v, thr_v, rows_v, chunks_v, candv_v, candi_v, topk_v,
          codes_v, sem):
        wid = lax.axis_index("s") * _NC + lax.axis_index("c")
        base = wid * npt
        tok0 = wid * _TPW
        pltpu.sync_copy(ids_hbm.at[pl.ds(base, npt)], ids_v)
        pltpu.sync_copy(thr_hbm.at[pl.ds(base, npt)], thr_v)

        iota = lax.iota(jnp.int32, _L)
        inf = jnp.float32(float("inf"))

        for g in range(nbat):
            def prep(u, _):
                s = g * 8 + u
                cidv = ids_v[pl.ds(_A * s, _A)]
                rows_v[g, pl.ds(_A * u, _A)] = cidv * 1024 + (tok0 + s)
                return 0
            lax.fori_loop(0, 8, prep, 0)

        for g in range(nbat):
            pltpu.async_copy(dist_hbm.at[rows_v.at[g]], chunks_v, sem).wait()

            def tok(u, _):
                s = g * 8 + u
                tvec = thr_v[pl.ds(_A * s, _A)]
                cidv = ids_v[pl.ds(_A * s, _A)]

                def scan(c, cnt):
                    cid128 = _splat_lane(cidv, c) * _CW
                    row = u * _A + c
                    for v in range(_CW // _L):
                        vals = chunks_v[row, pl.ds(_L * v, _L)]
                        mask = vals <= tvec
                        csum = plsc.cumsum(jnp.where(mask, 1, 0))
                        offs = cnt + csum - 1
                        gidx = cid128 + (v * _L + iota)
                        plsc.store_scatter(candv_v, [offs], vals, mask=mask)
                        plsc.store_scatter(candi_v, [offs], gidx, mask=mask)
                        cnt = cnt + _splat_lane(csum, _L - 1)
                    return cnt

                cntv = lax.fori_loop(0, _A, scan,
                                     jnp.zeros((_L,), jnp.int32))
                cnt = jnp.max(cntv)

                runk = jnp.full((_L,), inf)
                runv = jnp.zeros((_L,), jnp.int32)

                def merge(i, carry):
                    rk0, rv0 = carry
                    ck = candv_v[pl.ds(_L * i, _L)]
                    cv = candi_v[pl.ds(_L * i, _L)]
                    valid = (_L * i + iota) < cnt
                    ck = jnp.where(valid, ck, inf)
                    ck, cv = plsc.sort_key_val(ck, cv)
                    ckr = lax.rev(ck, (0,))
                    cvr = lax.rev(cv, (0,))
                    take = rk0 <= ckr
                    nk = jnp.where(take, rk0, ckr)
                    nv = jnp.where(take, rv0, cvr)
                    sk, sv = plsc.sort_key_val(nk, nv)
                    return (sk, sv)

                nv_ct = (cnt + _L - 1) // _L
                runk, runv = lax.fori_loop(0, nv_ct, merge, (runk, runv))
                topk_v[pl.ds(_A * s, _A)] = runv
                return 0

            lax.fori_loop(0, 8, tok, 0)

        pltpu.sync_copy(topk_v, topk_hbm.at[pl.ds(base, npt)])
        pltpu.async_copy(cb_hbm.at[topk_v], codes_v, sem).wait()
        pltpu.sync_copy(codes_v, codes_hbm.at[pl.ds(base, npt)])

    return k(dist_rows, ids_flat, thr_flat, cb_pad)


def _refine_body(nblk,
                 codes_ref, rrep_ref, xprep_ref, tki_ref,
                 w_in_ref, b_in_ref, w_h_ref, b_h_ref, w_out_ref, b_out_ref,
                 zq_ref, ind_ref, loss_ref, perp_ref, counts_ref):
    t = pl.program_id(0)

    @pl.when(t == 0)
    def _init():
        loss_ref[...] = jnp.zeros((1, 1), jnp.float32)
        perp_ref[...] = jnp.zeros((1, 1), jnp.float32)
        counts_ref[...] = jnp.zeros_like(counts_ref)

    codes = codes_ref[:, : _D]
    xp = xprep_ref[...]
    cx = jnp.concatenate([codes, xp], axis=1)
    h = cx @ w_in_ref[...] + b_in_ref[...]
    h = jnp.maximum(h, 0.0)
    for i in range(2):
        h = jnp.maximum(h @ w_h_ref[i] + b_h_ref[i][None, :], 0.0) + h
    delta = codes + h @ w_out_ref[...] + b_out_ref[...]

    diff = rrep_ref[...] - delta
    diff3 = diff.reshape(_BT, _A, _D)
    dist = jnp.sum(diff3 * diff3, axis=2)
    m = jnp.min(dist, axis=1, keepdims=True)
    ai = lax.broadcasted_iota(jnp.int32, dist.shape, 1)
    best = jnp.min(jnp.where(dist == m, ai, _A), axis=1)

    sel = ai == best[:, None]
    ind = jnp.sum(jnp.where(sel, tki_ref[...], 0), axis=1)
    ind_ref[...] = ind[:, None]

    zq = jnp.sum(delta.reshape(_BT, _A, _D)
                 * sel.astype(jnp.float32)[:, :, None], axis=1)
    zq_ref[...] = zq

    loss_ref[...] += jnp.sum(m).reshape(1, 1)
    onehot = (ind[:, None]
              == lax.broadcasted_iota(jnp.int32, (_BT, _NE), 1))
    counts_ref[...] += jnp.sum(onehot.astype(jnp.float32), axis=0,
                               keepdims=True)

    @pl.when(t == nblk - 1)
    def _fin():
        ntok = jnp.float32(nblk * _BT)
        p = counts_ref[...] / ntok
        ent = jnp.sum(p * jnp.log(p + 1e-10))
        perp_ref[...] = jnp.exp(-ent).reshape(1, 1)
        le = loss_ref[...] / (ntok * _D)
        loss_ref[...] = le + 0.25 * le


def _refine(codes_sel, r_rep, xp_rep, tki, params):
    B = tki.shape[0]
    nblk = B // _BT
    rb = _BT * _A
    out = pl.pallas_call(
        functools.partial(_refine_body, nblk),
        grid=(nblk,),
        in_specs=[
            pl.BlockSpec((rb, _CW), lambda t: (t, 0)),
            pl.BlockSpec((rb, _D), lambda t: (t, 0)),
            pl.BlockSpec((rb, _D), lambda t: (t, 0)),
            pl.BlockSpec((_BT, _A), lambda t: (t, 0)),
            pl.BlockSpec((2 * _D, _H), lambda t: (0, 0)),
            pl.BlockSpec((1, _H), lambda t: (0, 0)),
            pl.BlockSpec((2, _H, _H), lambda t: (0, 0, 0)),
            pl.BlockSpec((2, _H), lambda t: (0, 0)),
            pl.BlockSpec((_H, _D), lambda t: (0, 0)),
            pl.BlockSpec((1, _D), lambda t: (0, 0)),
        ],
        out_specs=[
            pl.BlockSpec((_BT, _D), lambda t: (t, 0)),
            pl.BlockSpec((_BT, 1), lambda t: (t, 0)),
            pl.BlockSpec((1, 1), lambda t: (0, 0)),
            pl.BlockSpec((1, 1), lambda t: (0, 0)),
        ],
        out_shape=[
            jax.ShapeDtypeStruct((B, _D), jnp.float32),
            jax.ShapeDtypeStruct((B, 1), jnp.int32),
            jax.ShapeDtypeStruct((1, 1), jnp.float32),
            jax.ShapeDtypeStruct((1, 1), jnp.float32),
        ],
        scratch_shapes=[pltpu.VMEM((1, _NE), jnp.float32)],
    )(codes_sel, r_rep, xp_rep, tki,
      params["W_in"], params["b_in"].reshape(1, _H),
      params["W_h"], params["b_h"],
      params["W_out"], params["b_out"].reshape(1, _D))
    return out


def kernel(residual, x_prev, codebook, params):
    orig_shape = residual.shape
    B = residual.size // _D
    r = residual.reshape(B, _D)
    xp = x_prev.reshape(B, _D)

    dist3, ids, thr16 = _dist_chunks(r, codebook)
    cb_pad = jnp.pad(codebook, ((0, 0), (0, _CW - _D)))
    tki_flat, codes_sel = _sc_topk_gather(
        dist3.reshape(_NCH * B, _CW), ids.reshape(-1), thr16.reshape(-1),
        cb_pad)
    tki = tki_flat.reshape(B, _A)
    r_rep = jnp.repeat(r, _A, axis=0)
    xp_rep = jnp.repeat(xp, _A, axis=0)
    zq, ind, loss, perp = _refine(codes_sel, r_rep, xp_rep, tki, params)

    return (zq.reshape(orig_shape), loss.reshape(()), perp.reshape(()),
            ind.reshape(-1))


# speedup vs baseline: 1.1443x; 1.1443x over previous
import functools

import jax
import jax.numpy as jnp
from jax import lax
from jax.experimental import pallas as pl
from jax.experimental.pallas import tpu as pltpu
from jax.experimental.pallas import tpu_sc as plsc

_NE = 8192
_D = 32
_H = 64
_A = 16
_BT = 256
_NCH = 64
_CW = 128
_NG = 8
_NC, _NS, _L = 2, 16, 16
_NW = _NC * _NS
_TPW = 1024 // _NW
_CAP = 2176


def _dist_chunks_body(r_ref, cb_ref, d3_ref, ids_ref, thr_ref, cm_ref):
    g = pl.program_id(0)
    r = r_ref[...]
    cb = cb_ref[...]
    rr = jnp.sum(r * r, axis=-1, keepdims=True)
    rc = lax.dot_general(r, cb, (((1,), (1,)), ((), ())),
                         preferred_element_type=jnp.float32)
    cc = jnp.sum(cb * cb, axis=-1)[None, :]
    d = (rr - 2.0 * rc) + cc
    mins = []
    for jj in range(_NG):
        slab = d[:, _CW * jj: _CW * (jj + 1)]
        d3_ref[jj] = slab
        mins.append(jnp.min(slab, axis=1, keepdims=True))
    cm_ref[g] = jnp.concatenate(mins, axis=1)

    @pl.when(g == _NG - 1)
    def _fin():
        cm = jnp.concatenate([cm_ref[i] for i in range(_NG)], axis=1)
        iota = lax.broadcasted_iota(jnp.int32, cm.shape, 1)
        inf = jnp.float32(float("inf"))
        cols = []
        m = None
        for _ in range(_A):
            m = jnp.min(cm, axis=1, keepdims=True)
            sel = jnp.min(jnp.where(cm == m, iota, _NCH), axis=1)
            cols.append(sel[:, None])
            cm = jnp.where(iota == sel[:, None], inf, cm)
        ids_ref[...] = jnp.concatenate(cols, axis=1)
        thr_ref[...] = jnp.broadcast_to(m, thr_ref.shape)


def _dist_chunks(r, cb):
    B = r.shape[0]
    return pl.pallas_call(
        _dist_chunks_body,
        grid=(_NG,),
        in_specs=[
            pl.BlockSpec((B, _D), lambda g: (0, 0)),
            pl.BlockSpec((_NE // _NG, _D), lambda g: (g, 0)),
        ],
        out_specs=[
            pl.BlockSpec((_NG, B, _CW), lambda g: (g, 0, 0)),
            pl.BlockSpec((B, _A), lambda g: (0, 0)),
            pl.BlockSpec((B, _A), lambda g: (0, 0)),
        ],
        out_shape=[
            jax.ShapeDtypeStruct((_NCH, B, _CW), jnp.float32),
            jax.ShapeDtypeStruct((B, _A), jnp.int32),
            jax.ShapeDtypeStruct((B, _A), jnp.float32),
        ],
        scratch_shapes=[pltpu.VMEM((_NG, B, _NG), jnp.float32)],
    )(r, cb)


def _splat_lane(vec, i):
    idx = jnp.full((_L,), i, jnp.int32)
    return lax.gather(
        vec, idx[:, None],
        lax.GatherDimensionNumbers(offset_dims=(), collapsed_slice_dims=(0,),
                                   start_index_map=(0,)),
        (1,), mode=lax.GatherScatterMode.PROMISE_IN_BOUNDS)


def _sc_topk_gather(dist_rows, ids_flat, thr_flat, cb_pad):
    mesh = plsc.VectorSubcoreMesh(core_axis_name="c", subcore_axis_name="s",
                                  num_cores=_NC, num_subcores=_NS)
    npt = _TPW * _A
    nbat = _TPW // 8

    @functools.partial(
        pl.kernel, mesh=mesh,
        compiler_params=pltpu.CompilerParams(needs_layout_passes=False),
        out_type=[
            jax.ShapeDtypeStruct((1024 * _A,), jnp.int32),
            jax.ShapeDtypeStruct((1024 * _A, _CW), jnp.float32),
        ],
        scratch_types=[
            pltpu.VMEM((npt,), jnp.int32),
            pltpu.VMEM((npt,), jnp.float32),
            pltpu.VMEM((nbat, 8 * _A), jnp.int32),
            pltpu.VMEM((8 * _A, _CW), jnp.float32),
            pltpu.VMEM((_CAP,), jnp.float32),
            pltpu.VMEM((_CAP,), jnp.int32),
            pltpu.VMEM((npt,), jnp.int32),
            pltpu.VMEM((npt, _CW), jnp.float32),
            pltpu.SemaphoreType.DMA,
        ],
    )
    def k(dist_hbm, ids_hbm, thr_hbm, cb_hbm, topk_hbm, codes_hbm,
          ids_v, thr_v, rows_v, chunks_v, candv_v, candi_v, topk_v,
          codes_v, sem):
        wid = lax.axis_index("s") * _NC + lax.axis_index("c")
        base = wid * npt
        tok0 = wid * _TPW
        pltpu.sync_copy(ids_hbm.at[pl.ds(base, npt)], ids_v)
        pltpu.sync_copy(thr_hbm.at[pl.ds(base, npt)], thr_v)

        iota = lax.iota(jnp.int32, _L)
        inf = jnp.float32(float("inf"))

        for g in range(nbat):
            def prep(u, _):
                s = g * 8 + u
                cidv = ids_v[pl.ds(_A * s, _A)]
                rows_v[g, pl.ds(_A * u, _A)] = cidv * 1024 + (tok0 + s)
                return 0
            lax.fori_loop(0, 8, prep, 0)

        for g in range(nbat):
            pltpu.async_copy(dist_hbm.at[rows_v.at[g]], chunks_v, sem).wait()

            def tok(u, _):
                s = g * 8 + u
                tvec = thr_v[pl.ds(_A * s, _A)]
                cidv = ids_v[pl.ds(_A * s, _A)]

                laneoff = iota * (_CAP // _L)

                def scan(c, pcnt):
                    cid128 = _splat_lane(cidv, c) * _CW
                    row = u * _A + c
                    for v in range(_CW // _L):
                        vals = chunks_v[row, pl.ds(_L * v, _L)]
                        mask = vals <= tvec
                        offs = laneoff + pcnt
                        gidx = cid128 + (v * _L + iota)
                        plsc.store_scatter(candv_v, [offs], vals, mask=mask)
                        plsc.store_scatter(candi_v, [offs], gidx, mask=mask)
                        pcnt = pcnt + jnp.where(mask, 1, 0)
                    return pcnt

                pcnt = lax.fori_loop(0, _A, scan,
                                     jnp.zeros((_L,), jnp.int32))
                kmax = jnp.max(pcnt)

                runk = jnp.full((_L,), inf)
                runv = jnp.zeros((_L,), jnp.int32)

                def merge(k2, carry):
                    rk0, rv0 = carry
                    idxs = laneoff + k2
                    ck = plsc.load_gather(candv_v, [idxs])
                    cv = plsc.load_gather(candi_v, [idxs])
                    ck = jnp.where(k2 < pcnt, ck, inf)
                    ck, cv = plsc.sort_key_val(ck, cv)
                    ckr = lax.rev(ck, (0,))
                    cvr = lax.rev(cv, (0,))
                    take = rk0 <= ckr
                    nk = jnp.where(take, rk0, ckr)
                    nv = jnp.where(take, rv0, cvr)
                    sk, sv = plsc.sort_key_val(nk, nv)
                    return (sk, sv)

                runk, runv = lax.fori_loop(0, kmax, merge, (runk, runv))
                topk_v[pl.ds(_A * s, _A)] = runv
                return 0

            lax.fori_loop(0, 8, tok, 0)

        pltpu.sync_copy(topk_v, topk_hbm.at[pl.ds(base, npt)])
        pltpu.async_copy(cb_hbm.at[topk_v], codes_v, sem).wait()
        pltpu.sync_copy(codes_v, codes_hbm.at[pl.ds(base, npt)])

    return k(dist_rows, ids_flat, thr_flat, cb_pad)


def _refine_body(nblk,
                 codes_ref, rrep_ref, xprep_ref, tki_ref,
                 w_in_ref, b_in_ref, w_h_ref, b_h_ref, w_out_ref, b_out_ref,
                 zq_ref, ind_ref, loss_ref, perp_ref, counts_ref):
    t = pl.program_id(0)

    @pl.when(t == 0)
    def _init():
        loss_ref[...] = jnp.zeros((1, 1), jnp.float32)
        perp_ref[...] = jnp.zeros((1, 1), jnp.float32)
        counts_ref[...] = jnp.zeros_like(counts_ref)

    codes = codes_ref[:, : _D]
    xp = xprep_ref[...]
    cx = jnp.concatenate([codes, xp], axis=1)
    h = cx @ w_in_ref[...] + b_in_ref[...]
    h = jnp.maximum(h, 0.0)
    for i in range(2):
        h = jnp.maximum(h @ w_h_ref[i] + b_h_ref[i][None, :], 0.0) + h
    delta = codes + h @ w_out_ref[...] + b_out_ref[...]

    diff = rrep_ref[...] - delta
    diff3 = diff.reshape(_BT, _A, _D)
    dist = jnp.sum(diff3 * diff3, axis=2)
    m = jnp.min(dist, axis=1, keepdims=True)
    ai = lax.broadcasted_iota(jnp.int32, dist.shape, 1)
    best = jnp.min(jnp.where(dist == m, ai, _A), axis=1)

    sel = ai == best[:, None]
    ind = jnp.sum(jnp.where(sel, tki_ref[...], 0), axis=1)
    ind_ref[...] = ind[:, None]

    zq = jnp.sum(delta.reshape(_BT, _A, _D)
                 * sel.astype(jnp.float32)[:, :, None], axis=1)
    zq_ref[...] = zq

    loss_ref[...] += jnp.sum(m).reshape(1, 1)
    onehot = (ind[:, None]
              == lax.broadcasted_iota(jnp.int32, (_BT, _NE), 1))
    counts_ref[...] += jnp.sum(onehot.astype(jnp.float32), axis=0,
                               keepdims=True)

    @pl.when(t == nblk - 1)
    def _fin():
        ntok = jnp.float32(nblk * _BT)
        p = counts_ref[...] / ntok
        ent = jnp.sum(p * jnp.log(p + 1e-10))
        perp_ref[...] = jnp.exp(-ent).reshape(1, 1)
        le = loss_ref[...] / (ntok * _D)
        loss_ref[...] = le + 0.25 * le


def _refine(codes_sel, r_rep, xp_rep, tki, params):
    B = tki.shape[0]
    nblk = B // _BT
    rb = _BT * _A
    out = pl.pallas_call(
        functools.partial(_refine_body, nblk),
        grid=(nblk,),
        in_specs=[
            pl.BlockSpec((rb, _CW), lambda t: (t, 0)),
            pl.BlockSpec((rb, _D), lambda t: (t, 0)),
            pl.BlockSpec((rb, _D), lambda t: (t, 0)),
            pl.BlockSpec((_BT, _A), lambda t: (t, 0)),
            pl.BlockSpec((2 * _D, _H), lambda t: (0, 0)),
            pl.BlockSpec((1, _H), lambda t: (0, 0)),
            pl.BlockSpec((2, _H, _H), lambda t: (0, 0, 0)),
            pl.BlockSpec((2, _H), lambda t: (0, 0)),
            pl.BlockSpec((_H, _D), lambda t: (0, 0)),
            pl.BlockSpec((1, _D), lambda t: (0, 0)),
        ],
        out_specs=[
            pl.BlockSpec((_BT, _D), lambda t: (t, 0)),
            pl.BlockSpec((_BT, 1), lambda t: (t, 0)),
            pl.BlockSpec((1, 1), lambda t: (0, 0)),
            pl.BlockSpec((1, 1), lambda t: (0, 0)),
        ],
        out_shape=[
            jax.ShapeDtypeStruct((B, _D), jnp.float32),
            jax.ShapeDtypeStruct((B, 1), jnp.int32),
            jax.ShapeDtypeStruct((1, 1), jnp.float32),
            jax.ShapeDtypeStruct((1, 1), jnp.float32),
        ],
        scratch_shapes=[pltpu.VMEM((1, _NE), jnp.float32)],
    )(codes_sel, r_rep, xp_rep, tki,
      params["W_in"], params["b_in"].reshape(1, _H),
      params["W_h"], params["b_h"],
      params["W_out"], params["b_out"].reshape(1, _D))
    return out


def kernel(residual, x_prev, codebook, params):
    orig_shape = residual.shape
    B = residual.size // _D
    r = residual.reshape(B, _D)
    xp = x_prev.reshape(B, _D)

    dist3, ids, thr16 = _dist_chunks(r, codebook)
    cb_pad = jnp.pad(codebook, ((0, 0), (0, _CW - _D)))
    tki_flat, codes_sel = _sc_topk_gather(
        dist3.reshape(_NCH * B, _CW), ids.reshape(-1), thr16.reshape(-1),
        cb_pad)
    tki = tki_flat.reshape(B, _A)
    r_rep = jnp.repeat(r, _A, axis=0)
    xp_rep = jnp.repeat(xp, _A, axis=0)
    zq, ind, loss, perp = _refine(codes_sel, r_rep, xp_rep, tki, params)

    return (zq.reshape(orig_shape), loss.reshape(()), perp.reshape(()),
            ind.reshape(-1))
